# baseline (speedup 1.0000x reference)
import functools

import jax
import jax.numpy as jnp
from jax import lax
from jax.experimental import pallas as pl
from jax.experimental.pallas import tpu as pltpu
from jax.experimental.pallas import tpu_sc as plsc

_D = 32
_L = 16
_NC = 2
_NS = 16
_NW = _NC * _NS
_TS = 8
_TB = 128
_CH = _TS * _TB


@functools.lru_cache(maxsize=None)
def _gather_call(bsz, seq):
    nst = seq // _TS
    ndt = _D // _TS
    nqg = _TB // _L
    mesh = plsc.VectorSubcoreMesh(core_axis_name="c", subcore_axis_name="s")

    @functools.partial(
        pl.kernel,
        mesh=mesh,
        out_type=jax.ShapeDtypeStruct((seq, ndt, _NW, _TS, _TB), jnp.float32),
        scratch_types=[
            pltpu.VMEM((2, _TS, _TB), jnp.int32),
            pltpu.VMEM((2, _CH, _D), jnp.float32),
            pltpu.VMEM((_TS, ndt, _TS, _TB), jnp.float32),
            (pltpu.SemaphoreType.DMA, pltpu.SemaphoreType.DMA),
        ],
        compiler_params=pltpu.CompilerParams(
            use_tc_tiling_on_sc=False, needs_layout_passes=False),
    )
    def k(idx4_hbm, tab_hbm, out5_hbm, idx_t, rows_v, trans_v, gsems):
        wid = lax.axis_index("s") * _NC + lax.axis_index("c")
        qvecs = [lax.iota(jnp.int32, _L) + qg * _L for qg in range(nqg)]
        divecs = [jnp.full((_L,), di, jnp.int32) for di in range(ndt)]
        dpvecs = [jnp.full((_L,), dp, jnp.int32) for dp in range(_TS)]

        def fire(g, b):
            pltpu.sync_copy(idx4_hbm.at[g, wid], idx_t.at[b])
            for p in range(_TS):
                pltpu.make_async_copy(
                    tab_hbm.at[idx_t.at[b, p]],
                    rows_v.at[b, pl.ds(p * _TB, _TB)],
                    gsems[b],
                ).start()

        def drain(b):
            for p in range(_TS):
                pltpu.make_async_copy(
                    tab_hbm.at[idx_t.at[b, p]],
                    rows_v.at[b, pl.ds(p * _TB, _TB)],
                    gsems[b],
                ).wait()

        def transpose_wb(g, b):
            def tbody(ps, tcarry):
                psvec = jnp.full((_L,), ps, jnp.int32)
                rowvecs = [psvec * _TB + qv for qv in qvecs]
                for di in range(ndt):
                    for dp in range(_TS):
                        dvec = jnp.full((_L,), di * _TS + dp, jnp.int32)
                        for qg in range(nqg):
                            vals = plsc.load_gather(
                                rows_v.at[b], [rowvecs[qg], dvec])
                            plsc.store_scatter(
                                trans_v,
                                [psvec, divecs[di], dpvecs[dp], qvecs[qg]],
                                vals)
                return tcarry

            lax.fori_loop(0, _TS, tbody, 0)
            pltpu.sync_copy(trans_v, out5_hbm.at[pl.ds(g * _TS, _TS), :, wid])

        fire(0, 0)

        def body(hb, carry):
            e = hb * 2
            fire(e + 1, 1)
            drain(0)
            transpose_wb(e, 0)
            fire(e + 2, 0)
            drain(1)
            transpose_wb(e + 1, 1)
            return carry

        lax.fori_loop(0, (nst - 1) // 2, body, 0)
        drain(0)
        transpose_wb(nst - 1, 0)

    return k


def kernel(x, w):
    bsz, seq = x.shape
    assert bsz == _NW * _TB and seq % (2 * _TS) == _TS and _D % _TS == 0
    nst = seq // _TS
    x4 = (x.astype(jnp.int32).T
          .reshape(nst, _TS, _NW, _TB).transpose(0, 2, 1, 3))
    out5 = _gather_call(bsz, seq)(x4, w)
    return out5.transpose(2, 4, 0, 1, 3).reshape(bsz, seq, _D)


# speedup vs baseline: 1.6022x; 1.6022x over previous
import functools

import jax
import jax.numpy as jnp
from jax import lax
from jax.experimental import pallas as pl
from jax.experimental.pallas import tpu as pltpu
from jax.experimental.pallas import tpu_sc as plsc

_D = 32
_L = 16
_NC = 2
_NS = 16
_NW = _NC * _NS
_TS = 8
_TB = 128
_CH = _TS * _TB


@functools.lru_cache(maxsize=None)
def _gather_call(bsz, seq):
    nst = seq // _TS
    ndt = _D // _TS
    nqg = _TB // _L
    mesh = plsc.VectorSubcoreMesh(core_axis_name="c", subcore_axis_name="s")

    @functools.partial(
        pl.kernel,
        mesh=mesh,
        out_type=jax.ShapeDtypeStruct((seq, ndt, _NW, _TS, _TB), jnp.float32),
        scratch_types=[
            pltpu.VMEM((2, _TS, _TB), jnp.int32),
            pltpu.VMEM((2, _CH, _D), jnp.float32),
            pltpu.VMEM((_TS, _D, _TB), jnp.float32),
            (pltpu.SemaphoreType.DMA, pltpu.SemaphoreType.DMA),
        ],
        compiler_params=pltpu.CompilerParams(
            use_tc_tiling_on_sc=False, needs_layout_passes=False),
    )
    def k(idx4_hbm, tab_hbm, out5_hbm, idx_t, rows_v, trans_v, gsems):
        wid = lax.axis_index("s") * _NC + lax.axis_index("c")
        iota = lax.iota(jnp.int32, _L)
        qvecs = [iota + qg * _L for qg in range(nqg)]

        def fire(g, b):
            pltpu.sync_copy(idx4_hbm.at[g, wid], idx_t.at[b])
            for p in range(_TS):
                pltpu.make_async_copy(
                    tab_hbm.at[idx_t.at[b, p]],
                    rows_v.at[b, pl.ds(p * _TB, _TB)],
                    gsems[b],
                ).start()

        def drain(b):
            for p in range(_TS):
                pltpu.make_async_copy(
                    tab_hbm.at[idx_t.at[b, p]],
                    rows_v.at[b, pl.ds(p * _TB, _TB)],
                    gsems[b],
                ).wait()

        def transpose_wb(g, b):
            def tbody(ps, tcarry):
                psvec = jnp.full((_L,), ps, jnp.int32)
                ps128 = jnp.full((_L,), ps * _TB, jnp.int32)
                rowvecs = [ps128 + qv for qv in qvecs]

                def cbody(c, ccarry):
                    dvecs = [(iota + c) & (_L - 1)]
                    for d0 in range(_L, _D, _L):
                        dvecs.append(dvecs[0] + d0)
                    for qg in range(nqg):
                        for dvec in dvecs:
                            vals = plsc.load_gather(
                                rows_v.at[b], [rowvecs[qg], dvec])
                            plsc.store_scatter(
                                trans_v, [psvec, dvec, qvecs[qg]], vals)
                    return ccarry

                lax.fori_loop(0, _L, cbody, 0)
                return tcarry

            lax.fori_loop(0, _TS, tbody, 0)
            for di in range(ndt):
                pltpu.sync_copy(
                    trans_v.at[:, pl.ds(di * _TS, _TS)],
                    out5_hbm.at[pl.ds(g * _TS, _TS), di, wid])

        fire(0, 0)

        def body(hb, carry):
            e = hb * 2
            fire(e + 1, 1)
            drain(0)
            transpose_wb(e, 0)
            fire(e + 2, 0)
            drain(1)
            transpose_wb(e + 1, 1)
            return carry

        lax.fori_loop(0, (nst - 1) // 2, body, 0)
        drain(0)
        transpose_wb(nst - 1, 0)

    return k


def kernel(x, w):
    bsz, seq = x.shape
    assert bsz == _NW * _TB and seq % (2 * _TS) == _TS and _D % _TS == 0
    nst = seq // _TS
    x4 = (x.astype(jnp.int32).T
          .reshape(nst, _TS, _NW, _TB).transpose(0, 2, 1, 3))
    out5 = _gather_call(bsz, seq)(x4, w)
    return out5.transpose(2, 4, 0, 1, 3).reshape(bsz, seq, _D)


# speedup vs baseline: 1.6514x; 1.0307x over previous
import functools

import jax
import jax.numpy as jnp
from jax import lax
from jax.experimental import pallas as pl
from jax.experimental.pallas import tpu as pltpu
from jax.experimental.pallas import tpu_sc as plsc

_D = 32
_L = 16
_NC = 2
_NS = 16
_NW = _NC * _NS
_TS = 8
_TB = 128
_CH = _TS * _TB


@functools.lru_cache(maxsize=None)
def _gather_call(bsz, seq):
    nst = seq // _TS
    ndt = _D // _TS
    nqg = _TB // _L
    mesh = plsc.VectorSubcoreMesh(core_axis_name="c", subcore_axis_name="s")

    @functools.partial(
        pl.kernel,
        mesh=mesh,
        out_type=jax.ShapeDtypeStruct((seq, ndt, _NW, _TS, _TB), jnp.float32),
        scratch_types=[
            pltpu.VMEM((2, _TS, _TB), jnp.int32),
            pltpu.VMEM((2, _CH, _D), jnp.float32),
            pltpu.VMEM((_TS, _D, _TB), jnp.float32),
            (pltpu.SemaphoreType.DMA, pltpu.SemaphoreType.DMA),
            pltpu.SemaphoreType.DMA,
        ],
        compiler_params=pltpu.CompilerParams(
            use_tc_tiling_on_sc=False, needs_layout_passes=False),
    )
    def k(idx4_hbm, tab_hbm, out5_hbm, idx_t, rows_v, trans_v, gsems, wsem):
        wid = lax.axis_index("s") * _NC + lax.axis_index("c")
        iota = lax.iota(jnp.int32, _L)
        qvecs = [iota + qg * _L for qg in range(nqg)]

        def fire(g, b):
            pltpu.sync_copy(idx4_hbm.at[g, wid], idx_t.at[b])
            for p in range(_TS):
                pltpu.make_async_copy(
                    tab_hbm.at[idx_t.at[b, p]],
                    rows_v.at[b, pl.ds(p * _TB, _TB)],
                    gsems[b],
                ).start()

        def drain(b):
            for p in range(_TS):
                pltpu.make_async_copy(
                    tab_hbm.at[idx_t.at[b, p]],
                    rows_v.at[b, pl.ds(p * _TB, _TB)],
                    gsems[b],
                ).wait()

        def wb_copies(g):
            return [
                pltpu.make_async_copy(
                    trans_v.at[:, pl.ds(di * _TS, _TS)],
                    out5_hbm.at[pl.ds(g * _TS, _TS), di, wid],
                    wsem,
                )
                for di in range(ndt)
            ]

        def transpose_wb(g, b):
            @pl.when(g > 0)
            def _():
                for c in wb_copies(0):
                    c.wait()

            def tbody(ps, tcarry):
                psvec = jnp.full((_L,), ps, jnp.int32)
                ps128 = jnp.full((_L,), ps * _TB, jnp.int32)
                rowvecs = [ps128 + qv for qv in qvecs]

                def cbody(c2, ccarry):
                    c = c2 * 2
                    dvecs = []
                    for dc in range(2):
                        d = [(iota + (c + dc)) & (_L - 1)]
                        for d0 in range(_L, _D, _L):
                            d.append(d[0] + d0)
                        dvecs += d
                    for qg in range(nqg):
                        for dvec in dvecs:
                            vals = plsc.load_gather(
                                rows_v.at[b], [rowvecs[qg], dvec])
                            plsc.store_scatter(
                                trans_v, [psvec, dvec, qvecs[qg]], vals)
                    return ccarry

                lax.fori_loop(0, _L // 2, cbody, 0)
                return tcarry

            lax.fori_loop(0, _TS, tbody, 0)
            for c in wb_copies(g):
                c.start()

        fire(0, 0)

        def body(hb, carry):
            e = hb * 2
            fire(e + 1, 1)
            drain(0)
            transpose_wb(e, 0)
            fire(e + 2, 0)
            drain(1)
            transpose_wb(e + 1, 1)
            return carry

        lax.fori_loop(0, (nst - 1) // 2, body, 0)
        drain(0)
        transpose_wb(nst - 1, 0)
        for c in wb_copies(0):
            c.wait()

    return k


def kernel(x, w):
    bsz, seq = x.shape
    assert bsz == _NW * _TB and seq % (2 * _TS) == _TS and _D % _TS == 0
    nst = seq // _TS
    x4 = (x.astype(jnp.int32).T
          .reshape(nst, _TS, _NW, _TB).transpose(0, 2, 1, 3))
    out5 = _gather_call(bsz, seq)(x4, w)
    return out5.transpose(2, 4, 0, 1, 3).reshape(bsz, seq, _D)
